# baseline (speedup 1.0000x reference)
import jax
import jax.numpy as jnp
from jax import lax
from jax.experimental import pallas as pl
from jax.experimental.pallas import tpu as pltpu
from jax.experimental.pallas import tpu_sc as plsc

N = 10000
E = 320000
D = 128
NC = 2
NS = 16
NW = NC * NS
ROWS = E // 128
NP = 10240
NPER = NP // NS
CPW = 80

import numpy as _np
_bounds = [w * ROWS // NW for w in range(NW + 1)]
_row_map = _np.concatenate([
    _np.concatenate([_np.arange(_bounds[w], _bounds[w + 1], dtype=_np.int32),
                     _np.full(CPW - (_bounds[w + 1] - _bounds[w]), -1,
                              _np.int32)])
    for w in range(NW)])
_REAL = _row_map >= 0
_SAFE = _np.where(_REAL, _row_map, 0).astype(_np.int32)
_TRASH = _np.repeat(N + _np.arange(NW, dtype=_np.int32), CPW)

_MESH = plsc.VectorSubcoreMesh(
    core_axis_name="c", subcore_axis_name="s", num_cores=NC, num_subcores=NS
)


def _make_sc_agg(with_cnt: bool):
  out_type = [jax.ShapeDtypeStruct((NC, NP, D), jnp.float32)]
  if with_cnt:
    out_type.append(jax.ShapeDtypeStruct((NC, NP), jnp.float32))

  scratch = [
      pltpu.VMEM((10, 128), jnp.int32),
      pltpu.VMEM((10, 128), jnp.int32),
      pltpu.VMEM((128, D), jnp.float32),
      pltpu.VMEM((128,), jnp.float32),
      pltpu.VMEM_SHARED((NP, D), jnp.float32),
      pltpu.VMEM_SHARED((NP,), jnp.float32),
      pltpu.SemaphoreType.DMA,
  ]

  def body(x_hbm, src_hbm, dst_hbm, zeros_hbm, zeros_n_hbm, ones_hbm, *rest):
    if with_cnt:
      agg_out, cnt_out = rest[0], rest[1]
      rest = rest[2:]
    else:
      agg_out, cnt_out = rest[0], None
      rest = rest[1:]
    sidx, didx, rows, ones_v, agg_sh, cnt_sh, sem = rest

    cid = lax.axis_index("c")
    sid = lax.axis_index("s")
    wid = sid * NC + cid
    lo = wid * CPW

    pltpu.sync_copy(zeros_hbm.at[pl.ds(sid * NPER, NPER)],
                    agg_sh.at[pl.ds(sid * NPER, NPER)])
    if with_cnt:
      pltpu.sync_copy(zeros_n_hbm.at[pl.ds(sid * NPER, NPER)],
                      cnt_sh.at[pl.ds(sid * NPER, NPER)])
      pltpu.sync_copy(ones_hbm, ones_v)
    plsc.subcore_barrier()

    def step(i, carry):
      pltpu.sync_copy(src_hbm.at[pl.ds(lo + i * 10, 10)], sidx)
      pltpu.sync_copy(dst_hbm.at[pl.ds(lo + i * 10, 10)], didx)
      for b in range(10):
        pltpu.async_copy(x_hbm.at[sidx.at[b]], rows, sem).wait()
        pltpu.sync_copy(rows, agg_sh.at[didx.at[b]], add=True)
        if with_cnt:
          pltpu.sync_copy(ones_v, cnt_sh.at[didx.at[b]], add=True)
      return carry

    lax.fori_loop(0, CPW // 10, step, 0)
    plsc.subcore_barrier()

    pltpu.sync_copy(agg_sh.at[pl.ds(sid * NPER, NPER)],
                    agg_out.at[cid, pl.ds(sid * NPER, NPER)])
    if with_cnt:
      pltpu.sync_copy(cnt_sh.at[pl.ds(sid * NPER, NPER)],
                      cnt_out.at[cid, pl.ds(sid * NPER, NPER)])

  return pl.kernel(body, out_type=tuple(out_type), mesh=_MESH,
                   scratch_types=scratch,
                   compiler_params=pltpu.CompilerParams(
                       use_tc_tiling_on_sc=False))


_sc_agg_cnt = _make_sc_agg(with_cnt=True)
_sc_agg = _make_sc_agg(with_cnt=False)

BN = 1000


def _tc_layer1_body(a0, a1, c0, c1, x, wl, wr, b, o):
  c = jnp.maximum(c0[...] + c1[...], 1.0)
  m = (a0[...] + a1[...]) / c
  acc = jnp.dot(m, wl[...], preferred_element_type=jnp.float32)
  acc += jnp.dot(x[...], wr[...], preferred_element_type=jnp.float32)
  o[...] = jnp.maximum(acc + b[...], 0.0)


def _tc_layer2_body(a0, a1, c0, c1, x, wl, wr, b, lw, lb, o):
  c = jnp.maximum(c0[...] + c1[...], 1.0)
  m = (a0[...] + a1[...]) / c
  acc = jnp.dot(m, wl[...], preferred_element_type=jnp.float32)
  acc += jnp.dot(x[...], wr[...], preferred_element_type=jnp.float32)
  h = jnp.maximum(acc + b[...], 0.0)
  o[...] = jnp.dot(h, lw[...], preferred_element_type=jnp.float32) + lb[...]


_ROW_SPEC = pl.BlockSpec((BN, D), lambda i: (i, 0))
_CNT_SPEC = pl.BlockSpec((BN, 1), lambda i: (i, 0))
_W_SPEC = pl.BlockSpec((D, D), lambda i: (0, 0))
_B_SPEC = pl.BlockSpec((1, D), lambda i: (0, 0))

_tc_layer1 = pl.pallas_call(
    _tc_layer1_body,
    grid=(N // BN,),
    in_specs=[_ROW_SPEC, _ROW_SPEC, _CNT_SPEC, _CNT_SPEC, _ROW_SPEC,
              _W_SPEC, _W_SPEC, _B_SPEC],
    out_specs=_ROW_SPEC,
    out_shape=jax.ShapeDtypeStruct((N, D), jnp.float32),
)

_tc_layer2 = pl.pallas_call(
    _tc_layer2_body,
    grid=(N // BN,),
    in_specs=[_ROW_SPEC, _ROW_SPEC, _CNT_SPEC, _CNT_SPEC, _ROW_SPEC,
              _W_SPEC, _W_SPEC, _B_SPEC,
              pl.BlockSpec((D, 1), lambda i: (0, 0)),
              pl.BlockSpec((1, 1), lambda i: (0, 0))],
    out_specs=pl.BlockSpec((BN, 1), lambda i: (i, 0)),
    out_shape=jax.ShapeDtypeStruct((N, 1), jnp.float32),
)


def kernel(x, edge_index, W1l, W1r, b1, W2l, W2r, b2, lin_W, lin_b):
  src2d = edge_index[0].reshape(ROWS, 128)
  dst2d = edge_index[1].reshape(ROWS, 128)
  real = jnp.asarray(_REAL)[:, None]
  src_r = jnp.where(real, src2d[jnp.asarray(_SAFE)], 0)
  dst_r = jnp.where(real, dst2d[jnp.asarray(_SAFE)],
                    jnp.asarray(_TRASH)[:, None])
  zeros = jnp.zeros((NP, D), jnp.float32)
  zeros_n = jnp.zeros((NP,), jnp.float32)
  ones = jnp.ones((128,), jnp.float32)

  agg1, cnt = _sc_agg_cnt(x, src_r, dst_r, zeros, zeros_n, ones)
  c0 = cnt[0, :N].reshape(N, 1)
  c1 = cnt[1, :N].reshape(N, 1)
  h1 = _tc_layer1(agg1[0, :N], agg1[1, :N], c0, c1, x, W1l, W1r,
                  b1.reshape(1, D))

  (agg2,) = _sc_agg(h1, src_r, dst_r, zeros, zeros_n, ones)
  out = _tc_layer2(agg2[0, :N], agg2[1, :N], c0, c1, h1, W2l, W2r,
                   b2.reshape(1, D), lin_W, lin_b.reshape(1, 1))
  return out


# speedup vs baseline: 1.7079x; 1.7079x over previous
import jax
import jax.numpy as jnp
from jax import lax
from jax.experimental import pallas as pl
from jax.experimental.pallas import tpu as pltpu
from jax.experimental.pallas import tpu_sc as plsc

N = 10000
E = 320000
D = 128
NC = 2
NS = 16
NW = NC * NS
ROWS = E // 128
NP = 10240
NPER = NP // NS
CPW = 80

import numpy as _np
_bounds = [w * ROWS // NW for w in range(NW + 1)]
_row_map = _np.concatenate([
    _np.concatenate([_np.arange(_bounds[w], _bounds[w + 1], dtype=_np.int32),
                     _np.full(CPW - (_bounds[w + 1] - _bounds[w]), -1,
                              _np.int32)])
    for w in range(NW)])
_REAL = _row_map >= 0
_SAFE = _np.where(_REAL, _row_map, 0).astype(_np.int32)
_rr = _np.arange(NW * CPW, dtype=_np.int32)[:, None]
_ll = _np.arange(128, dtype=_np.int32)[None, :]
_TRASH2D = (N + (_rr * 13 + _ll) % (NP - N)).astype(_np.int32)
_SRCDUMMY2D = ((_rr * 37 + _ll * 101) % N).astype(_np.int32)

_MESH = plsc.VectorSubcoreMesh(
    core_axis_name="c", subcore_axis_name="s", num_cores=NC, num_subcores=NS
)


def _make_sc_agg(with_cnt: bool):
  out_type = [jax.ShapeDtypeStruct((NC, NP, D), jnp.float32)]
  if with_cnt:
    out_type.append(jax.ShapeDtypeStruct((NC, NP), jnp.float32))

  scratch = [
      pltpu.VMEM((128,), jnp.int32),
      pltpu.VMEM((128,), jnp.int32),
      pltpu.VMEM((128, D), jnp.float32),
      pltpu.VMEM((128,), jnp.float32),
      pltpu.VMEM_SHARED((NP, D), jnp.float32),
      pltpu.VMEM_SHARED((NP,), jnp.float32),
      pltpu.SemaphoreType.DMA,
  ]

  def body(x_hbm, src_hbm, dst_hbm, zeros_hbm, zeros_n_hbm, ones_hbm, *rest):
    if with_cnt:
      agg_out, cnt_out = rest[0], rest[1]
      rest = rest[2:]
    else:
      agg_out, cnt_out = rest[0], None
      rest = rest[1:]
    sidx, didx, rows, ones_v, agg_sh, cnt_sh, sem = rest

    cid = lax.axis_index("c")
    sid = lax.axis_index("s")
    wid = sid * NC + cid
    lo = wid * CPW

    pltpu.sync_copy(zeros_hbm.at[pl.ds(sid * NPER, NPER)],
                    agg_sh.at[pl.ds(sid * NPER, NPER)])
    if with_cnt:
      pltpu.sync_copy(zeros_n_hbm.at[pl.ds(sid * NPER, NPER)],
                      cnt_sh.at[pl.ds(sid * NPER, NPER)])
      pltpu.sync_copy(ones_hbm, ones_v)
    plsc.subcore_barrier()

    def step(j, carry):
      pltpu.sync_copy(src_hbm.at[pl.ds((lo + j) * 128, 128)], sidx)
      pltpu.sync_copy(dst_hbm.at[pl.ds((lo + j) * 128, 128)], didx)
      pltpu.async_copy(x_hbm.at[sidx], rows, sem).wait()
      pltpu.sync_copy(rows, agg_sh.at[didx], add=True)
      if with_cnt:
        pltpu.sync_copy(ones_v, cnt_sh.at[didx], add=True)
      return carry

    lax.fori_loop(0, CPW, step, 0)
    plsc.subcore_barrier()

    pltpu.sync_copy(agg_sh.at[pl.ds(sid * NPER, NPER)],
                    agg_out.at[cid, pl.ds(sid * NPER, NPER)])
    if with_cnt:
      pltpu.sync_copy(cnt_sh.at[pl.ds(sid * NPER, NPER)],
                      cnt_out.at[cid, pl.ds(sid * NPER, NPER)])

  return pl.kernel(body, out_type=tuple(out_type), mesh=_MESH,
                   scratch_types=scratch,
                   compiler_params=pltpu.CompilerParams(
                       use_tc_tiling_on_sc=False))


_sc_agg_cnt = _make_sc_agg(with_cnt=True)
_sc_agg = _make_sc_agg(with_cnt=False)

BN = 1000


def _tc_layer1_body(a0, a1, c0, c1, x, wl, wr, b, o):
  c = jnp.maximum(c0[...] + c1[...], 1.0)
  m = (a0[...] + a1[...]) / c
  acc = jnp.dot(m, wl[...], preferred_element_type=jnp.float32)
  acc += jnp.dot(x[...], wr[...], preferred_element_type=jnp.float32)
  o[...] = jnp.maximum(acc + b[...], 0.0)


def _tc_layer2_body(a0, a1, c0, c1, x, wl, wr, b, lw, lb, o):
  c = jnp.maximum(c0[...] + c1[...], 1.0)
  m = (a0[...] + a1[...]) / c
  acc = jnp.dot(m, wl[...], preferred_element_type=jnp.float32)
  acc += jnp.dot(x[...], wr[...], preferred_element_type=jnp.float32)
  h = jnp.maximum(acc + b[...], 0.0)
  o[...] = jnp.dot(h, lw[...], preferred_element_type=jnp.float32) + lb[...]


_ROW_SPEC = pl.BlockSpec((BN, D), lambda i: (i, 0))
_CNT_SPEC = pl.BlockSpec((BN, 1), lambda i: (i, 0))
_W_SPEC = pl.BlockSpec((D, D), lambda i: (0, 0))
_B_SPEC = pl.BlockSpec((1, D), lambda i: (0, 0))

_tc_layer1 = pl.pallas_call(
    _tc_layer1_body,
    grid=(N // BN,),
    in_specs=[_ROW_SPEC, _ROW_SPEC, _CNT_SPEC, _CNT_SPEC, _ROW_SPEC,
              _W_SPEC, _W_SPEC, _B_SPEC],
    out_specs=_ROW_SPEC,
    out_shape=jax.ShapeDtypeStruct((N, D), jnp.float32),
)

_tc_layer2 = pl.pallas_call(
    _tc_layer2_body,
    grid=(N // BN,),
    in_specs=[_ROW_SPEC, _ROW_SPEC, _CNT_SPEC, _CNT_SPEC, _ROW_SPEC,
              _W_SPEC, _W_SPEC, _B_SPEC,
              pl.BlockSpec((D, 1), lambda i: (0, 0)),
              pl.BlockSpec((1, 1), lambda i: (0, 0))],
    out_specs=pl.BlockSpec((BN, 1), lambda i: (i, 0)),
    out_shape=jax.ShapeDtypeStruct((N, 1), jnp.float32),
)


def kernel(x, edge_index, W1l, W1r, b1, W2l, W2r, b2, lin_W, lin_b):
  src2d = edge_index[0].reshape(ROWS, 128)
  dst2d = edge_index[1].reshape(ROWS, 128)
  real = jnp.asarray(_REAL)[:, None]
  src_r = jnp.where(real, src2d[jnp.asarray(_SAFE)],
                    jnp.asarray(_SRCDUMMY2D)).reshape(-1)
  dst_r = jnp.where(real, dst2d[jnp.asarray(_SAFE)],
                    jnp.asarray(_TRASH2D)).reshape(-1)
  zeros = jnp.zeros((NP, D), jnp.float32)
  zeros_n = jnp.zeros((NP,), jnp.float32)
  ones = jnp.ones((128,), jnp.float32)

  agg1, cnt = _sc_agg_cnt(x, src_r, dst_r, zeros, zeros_n, ones)
  c0 = cnt[0, :N].reshape(N, 1)
  c1 = cnt[1, :N].reshape(N, 1)
  h1 = _tc_layer1(agg1[0, :N], agg1[1, :N], c0, c1, x, W1l, W1r,
                  b1.reshape(1, D))

  (agg2,) = _sc_agg(h1, src_r, dst_r, zeros, zeros_n, ones)
  out = _tc_layer2(agg2[0, :N], agg2[1, :N], c0, c1, h1, W2l, W2r,
                   b2.reshape(1, D), lin_W, lin_b.reshape(1, 1))
  return out


# speedup vs baseline: 2.5410x; 1.4878x over previous
import jax
import jax.numpy as jnp
from jax import lax
from jax.experimental import pallas as pl
from jax.experimental.pallas import tpu as pltpu
from jax.experimental.pallas import tpu_sc as plsc

N = 10000
E = 320000
D = 128
NC = 2
NS = 16
NW = NC * NS
ROWS = E // 128
NP = 10240
NPER = NP // NS
CPW = 80

import numpy as _np
_bounds = [w * ROWS // NW for w in range(NW + 1)]
_row_map = _np.concatenate([
    _np.concatenate([_np.arange(_bounds[w], _bounds[w + 1], dtype=_np.int32),
                     _np.full(CPW - (_bounds[w + 1] - _bounds[w]), -1,
                              _np.int32)])
    for w in range(NW)])
_REAL = _row_map >= 0
_SAFE = _np.where(_REAL, _row_map, 0).astype(_np.int32)
_rr = _np.arange(NW * CPW, dtype=_np.int32)[:, None]
_ll = _np.arange(128, dtype=_np.int32)[None, :]
_TRASH2D = (N + (_rr * 13 + _ll) % (NP - N)).astype(_np.int32)
_SRCDUMMY2D = ((_rr * 37 + _ll * 101) % N).astype(_np.int32)

_MESH = plsc.VectorSubcoreMesh(
    core_axis_name="c", subcore_axis_name="s", num_cores=NC, num_subcores=NS
)


def _make_sc_agg(with_cnt: bool):
  out_type = [jax.ShapeDtypeStruct((NC, NP, D), jnp.float32)]
  if with_cnt:
    out_type.append(jax.ShapeDtypeStruct((NC, NP), jnp.float32))

  scratch = [
      pltpu.VMEM((128,), jnp.int32),
      pltpu.VMEM((128,), jnp.int32),
      pltpu.VMEM((128,), jnp.int32),
      pltpu.VMEM((128,), jnp.int32),
      pltpu.VMEM((128, D), jnp.float32),
      pltpu.VMEM((128, D), jnp.float32),
      pltpu.VMEM((128,), jnp.float32),
      pltpu.VMEM_SHARED((NP, D), jnp.float32),
      pltpu.VMEM_SHARED((NP,), jnp.float32),
  ] + [pltpu.SemaphoreType.DMA] * 3

  def body(x_hbm, src_hbm, dst_hbm, zeros_hbm, zeros_n_hbm, ones_hbm, *rest):
    if with_cnt:
      agg_out, cnt_out = rest[0], rest[1]
      rest = rest[2:]
    else:
      agg_out, cnt_out = rest[0], None
      rest = rest[1:]
    (sidx0, sidx1, didx0, didx1, rows0, rows1, ones_v, agg_sh, cnt_sh,
     g0, g1, csem) = rest
    sidx = (sidx0, sidx1)
    didx = (didx0, didx1)
    rows = (rows0, rows1)
    g_sems = (g0, g1)

    cid = lax.axis_index("c")
    sid = lax.axis_index("s")
    wid = sid * NC + cid
    lo = wid * CPW

    pltpu.sync_copy(zeros_hbm.at[pl.ds(sid * NPER, NPER)],
                    agg_sh.at[pl.ds(sid * NPER, NPER)])
    if with_cnt:
      pltpu.sync_copy(zeros_n_hbm.at[pl.ds(sid * NPER, NPER)],
                      cnt_sh.at[pl.ds(sid * NPER, NPER)])
      pltpu.sync_copy(ones_hbm, ones_v)
    plsc.subcore_barrier()

    def stage(j, b):
      pltpu.sync_copy(src_hbm.at[pl.ds((lo + j) * 128, 128)], sidx[b])
      pltpu.sync_copy(dst_hbm.at[pl.ds((lo + j) * 128, 128)], didx[b])

    def fire_gather(b):
      pltpu.async_copy(x_hbm.at[sidx[b]], rows[b], g_sems[b])

    def wait_gather(b):
      pltpu.make_async_copy(x_hbm.at[sidx[b]], rows[b], g_sems[b]).wait()

    def scatter(b):
      if with_cnt:
        pltpu.async_copy(ones_v, cnt_sh.at[didx[b]], csem, add=True)
      pltpu.sync_copy(rows[b], agg_sh.at[didx[b]], add=True)

    def wait_cnt():
      if with_cnt:
        pltpu.make_async_copy(ones_v, cnt_sh.at[didx0], csem).wait()

    stage(0, 0)
    fire_gather(0)

    def pair(i, carry):
      j0 = 2 * i
      stage(j0 + 1, 1)
      fire_gather(1)
      wait_gather(0)
      scatter(0)
      wait_cnt()
      stage(j0 + 2, 0)
      fire_gather(0)
      wait_gather(1)
      scatter(1)
      wait_cnt()
      return carry

    lax.fori_loop(0, CPW // 2 - 1, pair, 0)

    stage(CPW - 1, 1)
    fire_gather(1)
    wait_gather(0)
    scatter(0)
    wait_cnt()
    wait_gather(1)
    scatter(1)
    wait_cnt()
    plsc.subcore_barrier()

    pltpu.sync_copy(agg_sh.at[pl.ds(sid * NPER, NPER)],
                    agg_out.at[cid, pl.ds(sid * NPER, NPER)])
    if with_cnt:
      pltpu.sync_copy(cnt_sh.at[pl.ds(sid * NPER, NPER)],
                      cnt_out.at[cid, pl.ds(sid * NPER, NPER)])

  return pl.kernel(body, out_type=tuple(out_type), mesh=_MESH,
                   scratch_types=scratch,
                   compiler_params=pltpu.CompilerParams(
                       use_tc_tiling_on_sc=False))


_sc_agg_cnt = _make_sc_agg(with_cnt=True)
_sc_agg = _make_sc_agg(with_cnt=False)

BN = 1000


def _tc_layer1_body(a0, a1, c0, c1, x, wl, wr, b, o):
  c = jnp.maximum(c0[...] + c1[...], 1.0)
  m = (a0[...] + a1[...]) / c
  acc = jnp.dot(m, wl[...], preferred_element_type=jnp.float32)
  acc += jnp.dot(x[...], wr[...], preferred_element_type=jnp.float32)
  o[...] = jnp.maximum(acc + b[...], 0.0)


def _tc_layer2_body(a0, a1, c0, c1, x, wl, wr, b, lw, lb, o):
  c = jnp.maximum(c0[...] + c1[...], 1.0)
  m = (a0[...] + a1[...]) / c
  acc = jnp.dot(m, wl[...], preferred_element_type=jnp.float32)
  acc += jnp.dot(x[...], wr[...], preferred_element_type=jnp.float32)
  h = jnp.maximum(acc + b[...], 0.0)
  o[...] = jnp.dot(h, lw[...], preferred_element_type=jnp.float32) + lb[...]


_ROW_SPEC = pl.BlockSpec((BN, D), lambda i: (i, 0))
_CNT_SPEC = pl.BlockSpec((BN, 1), lambda i: (i, 0))
_W_SPEC = pl.BlockSpec((D, D), lambda i: (0, 0))
_B_SPEC = pl.BlockSpec((1, D), lambda i: (0, 0))

_tc_layer1 = pl.pallas_call(
    _tc_layer1_body,
    grid=(N // BN,),
    in_specs=[_ROW_SPEC, _ROW_SPEC, _CNT_SPEC, _CNT_SPEC, _ROW_SPEC,
              _W_SPEC, _W_SPEC, _B_SPEC],
    out_specs=_ROW_SPEC,
    out_shape=jax.ShapeDtypeStruct((N, D), jnp.float32),
)

_tc_layer2 = pl.pallas_call(
    _tc_layer2_body,
    grid=(N // BN,),
    in_specs=[_ROW_SPEC, _ROW_SPEC, _CNT_SPEC, _CNT_SPEC, _ROW_SPEC,
              _W_SPEC, _W_SPEC, _B_SPEC,
              pl.BlockSpec((D, 1), lambda i: (0, 0)),
              pl.BlockSpec((1, 1), lambda i: (0, 0))],
    out_specs=pl.BlockSpec((BN, 1), lambda i: (i, 0)),
    out_shape=jax.ShapeDtypeStruct((N, 1), jnp.float32),
)


def kernel(x, edge_index, W1l, W1r, b1, W2l, W2r, b2, lin_W, lin_b):
  src2d = edge_index[0].reshape(ROWS, 128)
  dst2d = edge_index[1].reshape(ROWS, 128)
  real = jnp.asarray(_REAL)[:, None]
  src_r = jnp.where(real, src2d[jnp.asarray(_SAFE)],
                    jnp.asarray(_SRCDUMMY2D)).reshape(-1)
  dst_r = jnp.where(real, dst2d[jnp.asarray(_SAFE)],
                    jnp.asarray(_TRASH2D)).reshape(-1)
  zeros = jnp.zeros((NP, D), jnp.float32)
  zeros_n = jnp.zeros((NP,), jnp.float32)
  ones = jnp.ones((128,), jnp.float32)

  agg1, cnt = _sc_agg_cnt(x, src_r, dst_r, zeros, zeros_n, ones)
  c0 = cnt[0, :N].reshape(N, 1)
  c1 = cnt[1, :N].reshape(N, 1)
  h1 = _tc_layer1(agg1[0, :N], agg1[1, :N], c0, c1, x, W1l, W1r,
                  b1.reshape(1, D))

  (agg2,) = _sc_agg(h1, src_r, dst_r, zeros, zeros_n, ones)
  out = _tc_layer2(agg2[0, :N], agg2[1, :N], c0, c1, h1, W2l, W2r,
                   b2.reshape(1, D), lin_W, lin_b.reshape(1, 1))
  return out
